# Initial kernel scaffold; baseline (speedup 1.0000x reference)
import jax
import jax.numpy as jnp
from jax.experimental import pallas as pl


def kernel(x, edge_index, edge_attr, auxiliary, W1, b1, W2, b2, W_ih, b_ih, W_hh, b_hh):
    raise NotImplementedError("write your pallas kernel here")


# speedup vs baseline: 2.1531x; 2.1531x over previous
import functools

import jax
import jax.numpy as jnp
from jax import lax
from jax.experimental import pallas as pl
from jax.experimental.pallas import tpu as pltpu
from jax.experimental.pallas import tpu_sc as plsc

N_NODES = 10000
N_EDGES = 160000
D = 128
NPAD = 10240

NC, NS = 2, 16
NW = NC * NS
CH = 128
NCHUNK = N_EDGES // CH
RPT = NPAD // NS

def _worker_id():
    return lax.axis_index("s") * NC + lax.axis_index("c")


def _num_chunks(wid):
    base = NCHUNK // NW
    return base + (wid < (NCHUNK - base * NW)).astype(jnp.int32)


def _gather_body(h_hbm, dst_hbm, src_hbm, hd_hbm, hs_hbm,
                 idx_d, rows_d, idx_s, rows_s, sem_d, sem_s):
    wid = _worker_id()
    nk = _num_chunks(wid)

    def step(j, carry):
        off = (j * NW + wid) * CH
        pltpu.sync_copy(dst_hbm.at[pl.ds(off, CH)], idx_d)
        cp_d = pltpu.async_copy(h_hbm.at[idx_d], rows_d, sem_d)
        pltpu.sync_copy(src_hbm.at[pl.ds(off, CH)], idx_s)
        cp_s = pltpu.async_copy(h_hbm.at[idx_s], rows_s, sem_s)
        cp_d.wait()
        pltpu.sync_copy(rows_d, hd_hbm.at[pl.ds(off, CH)])
        cp_s.wait()
        pltpu.sync_copy(rows_s, hs_hbm.at[pl.ds(off, CH)])
        return carry

    lax.fori_loop(0, nk, step, 0)


@functools.lru_cache(maxsize=None)
def _build_gather():
    return pl.kernel(
        _gather_body,
        out_type=[jax.ShapeDtypeStruct((N_EDGES, D), jnp.float32),
                  jax.ShapeDtypeStruct((N_EDGES, D), jnp.float32)],
        mesh=plsc.VectorSubcoreMesh(core_axis_name="c", subcore_axis_name="s"),
        scratch_types=[pltpu.VMEM((CH,), jnp.int32),
                       pltpu.VMEM((CH, D), jnp.float32),
                       pltpu.VMEM((CH,), jnp.int32),
                       pltpu.VMEM((CH, D), jnp.float32),
                       pltpu.SemaphoreType.DMA,
                       pltpu.SemaphoreType.DMA],
    )


def _gather(h, dst, src):
    return _build_gather()(h, dst, src)


def _scatter_body(msg_hbm, dst_hbm, zeros_hbm, out_hbm, idx_v, rows_v, acc_sh):
    c = lax.axis_index("c")
    s = lax.axis_index("s")
    wid = _worker_id()
    nk = _num_chunks(wid)

    pltpu.sync_copy(zeros_hbm.at[pl.ds(s * RPT, RPT)],
                    acc_sh.at[pl.ds(s * RPT, RPT)])
    plsc.subcore_barrier()

    def step(j, carry):
        off = (j * NW + wid) * CH
        pltpu.sync_copy(msg_hbm.at[pl.ds(off, CH)], rows_v)
        pltpu.sync_copy(dst_hbm.at[pl.ds(off, CH)], idx_v)
        pltpu.sync_copy(rows_v, acc_sh.at[idx_v], add=True)
        return carry

    lax.fori_loop(0, nk, step, 0)
    plsc.subcore_barrier()
    pltpu.sync_copy(acc_sh.at[pl.ds(s * RPT, RPT)],
                    out_hbm.at[c, pl.ds(s * RPT, RPT)])


@functools.lru_cache(maxsize=None)
def _build_scatter():
    return pl.kernel(
        _scatter_body,
        out_type=[jax.ShapeDtypeStruct((NC, NPAD, D), jnp.float32)],
        mesh=plsc.VectorSubcoreMesh(core_axis_name="c", subcore_axis_name="s"),
        scratch_types=[pltpu.VMEM((CH,), jnp.int32),
                       pltpu.VMEM((CH, D), jnp.float32),
                       pltpu.VMEM_SHARED((NPAD, D), jnp.float32)],
    )


def _scatter(msg, dst, zeros_pad):
    return _build_scatter()(msg, dst, zeros_pad)


BE = 1280


def _mlp_body(hd_ref, hs_ref, ea_ref, ax_ref, W1_ref, b1_ref, W2_ref, b2_ref,
              out_ref):
    acc = jnp.dot(hd_ref[...], W1_ref[0:D, :],
                  preferred_element_type=jnp.float32)
    acc += jnp.dot(hs_ref[...], W1_ref[D:2 * D, :],
                   preferred_element_type=jnp.float32)
    acc += jnp.dot(ea_ref[...], W1_ref[2 * D:3 * D, :],
                   preferred_element_type=jnp.float32)
    acc += jnp.dot(ax_ref[...], W1_ref[3 * D:, :],
                   preferred_element_type=jnp.float32)
    acc += b1_ref[...]
    hdn = jnp.maximum(acc, 0.0)
    out_ref[...] = jnp.dot(hdn, W2_ref[...],
                           preferred_element_type=jnp.float32) + b2_ref[...]


def _edge_mlp(hd, hs, ea, ax, W1r, b1r, W2r, b2r):
    n_in = 2 * D + ea.shape[1] + ax.shape[1]
    hid = W1r.shape[1]
    grid = N_EDGES // BE
    return pl.pallas_call(
        _mlp_body,
        grid=(grid,),
        in_specs=[
            pl.BlockSpec((BE, D), lambda i: (i, 0)),
            pl.BlockSpec((BE, D), lambda i: (i, 0)),
            pl.BlockSpec((BE, ea.shape[1]), lambda i: (i, 0)),
            pl.BlockSpec((BE, ax.shape[1]), lambda i: (i, 0)),
            pl.BlockSpec((n_in, hid), lambda i: (0, 0)),
            pl.BlockSpec((1, hid), lambda i: (0, 0)),
            pl.BlockSpec((hid, D), lambda i: (0, 0)),
            pl.BlockSpec((1, D), lambda i: (0, 0)),
        ],
        out_specs=pl.BlockSpec((BE, D), lambda i: (i, 0)),
        out_shape=jax.ShapeDtypeStruct((N_EDGES, D), jnp.float32),
    )(hd, hs, ea, ax, W1r, b1r, W2r, b2r)


BN = 2000


def _gru_body(ap_ref, h_ref, Wih_ref, bih_ref, Whh_ref, bhh_ref, out_ref):
    a = ap_ref[0] + ap_ref[1]
    h = h_ref[...]
    gi = lax.dot_general(a, Wih_ref[...], (((1,), (1,)), ((), ())),
                         preferred_element_type=jnp.float32) + bih_ref[...]
    gh = lax.dot_general(h, Whh_ref[...], (((1,), (1,)), ((), ())),
                         preferred_element_type=jnp.float32) + bhh_ref[...]
    r = jax.nn.sigmoid(gi[:, 0:D] + gh[:, 0:D])
    z = jax.nn.sigmoid(gi[:, D:2 * D] + gh[:, D:2 * D])
    n = jnp.tanh(gi[:, 2 * D:3 * D] + r * gh[:, 2 * D:3 * D])
    out_ref[...] = (1.0 - z) * n + z * h


def _gru(ap, h, Wihr, bihr, Whhr, bhhr):
    grid = N_NODES // BN
    return pl.pallas_call(
        _gru_body,
        grid=(grid,),
        in_specs=[
            pl.BlockSpec((NC, BN, D), lambda i: (0, i, 0)),
            pl.BlockSpec((BN, D), lambda i: (i, 0)),
            pl.BlockSpec((3 * D, D), lambda i: (0, 0)),
            pl.BlockSpec((1, 3 * D), lambda i: (0, 0)),
            pl.BlockSpec((3 * D, D), lambda i: (0, 0)),
            pl.BlockSpec((1, 3 * D), lambda i: (0, 0)),
        ],
        out_specs=pl.BlockSpec((BN, D), lambda i: (i, 0)),
        out_shape=jax.ShapeDtypeStruct((N_NODES, D), jnp.float32),
    )(ap, h, Wihr, bihr, Whhr, bhhr)


def kernel(x, edge_index, edge_attr, auxiliary, W1, b1, W2, b2,
           W_ih, b_ih, W_hh, b_hh):
    ei = edge_index.astype(jnp.int32)
    src = ei[0]
    dst = ei[1]
    zeros_pad = jnp.zeros((NPAD, D), jnp.float32)
    h = x
    for r in range(W1.shape[0]):
        hd, hs = _gather(h, dst, src)
        msg = _edge_mlp(hd, hs, edge_attr, auxiliary,
                        W1[r], b1[r].reshape(1, -1), W2[r], b2[r].reshape(1, -1))
        (ap,) = _scatter(msg, dst, zeros_pad)
        h = _gru(ap, h, W_ih[r], b_ih[r].reshape(1, -1),
                 W_hh[r], b_hh[r].reshape(1, -1))
    return h
